# baseline (speedup 1.0000x reference)
import functools

import jax
import jax.numpy as jnp
from jax import lax
from jax.experimental import pallas as pl
from jax.experimental.pallas import tpu as pltpu
from jax.experimental.pallas import tpu_sc as plsc

B = 8
E = 96 * 224 * 224
N = B * E
NC, NS, L = 2, 16, 16
NW = NC * NS
WPS = NW // B
EP = E // WPS
CH = 24576
NCH = EP // CH
UNROLL = 8

_mesh = plsc.VectorSubcoreMesh(
    core_axis_name="c", subcore_axis_name="s", num_cores=NC, num_subcores=NS
)


def _worker_id():
    return lax.axis_index("s") * NC + lax.axis_index("c")


@functools.partial(
    pl.kernel,
    out_type=jax.ShapeDtypeStruct((NW, 3, L), jnp.float32),
    mesh=_mesh,
    scratch_types=[
        pltpu.VMEM((CH,), jnp.float32),
        pltpu.VMEM((3, L), jnp.float32),
    ],
)
def _stats_kernel(x_hbm, part_hbm, buf, part_v):
    wid = _worker_id()
    base = wid * EP

    zero = jnp.zeros((L,), jnp.float32)

    def chunk_body(ci, acc):
        pltpu.sync_copy(x_hbm.at[pl.ds(base + ci * CH, CH)], buf)

        def vec_body(i, c):
            n, s1, s2 = c
            v = buf[pl.ds(i, L)]
            m = jnp.maximum(v, 0.0)
            n = n + jnp.where(v >= 0.0, 1.0, 0.0)
            s1 = s1 + m
            s2 = s2 + m * m
            return (n, s1, s2)

        return plsc.parallel_loop(0, CH, L, unroll=UNROLL, carry=acc)(vec_body)

    n, s1, s2 = lax.fori_loop(0, NCH, chunk_body, (zero, zero, zero))
    part_v[0, :] = n
    part_v[1, :] = s1
    part_v[2, :] = s2
    pltpu.sync_copy(part_v, part_hbm.at[wid])


@functools.partial(
    pl.kernel,
    out_type=jax.ShapeDtypeStruct((N,), jnp.float32),
    mesh=_mesh,
    scratch_types=[
        pltpu.VMEM((CH,), jnp.float32),
        pltpu.VMEM((NW, 3, L), jnp.float32),
    ],
)
def _norm_kernel(x_hbm, part_hbm, out_hbm, buf, part_v):
    wid = _worker_id()
    b = wid // WPS
    base = wid * EP

    pltpu.sync_copy(part_hbm, part_v)
    nv = jnp.zeros((L,), jnp.float32)
    s1v = jnp.zeros((L,), jnp.float32)
    s2v = jnp.zeros((L,), jnp.float32)
    for k in range(WPS):
        w = b * WPS + k
        nv = nv + part_v[w, 0, :]
        s1v = s1v + part_v[w, 1, :]
        s2v = s2v + part_v[w, 2, :]
    n = nv[0]
    s1 = s1v[0]
    s2 = s2v[0]
    for j in range(1, L):
        n = n + nv[j]
        s1 = s1 + s1v[j]
        s2 = s2 + s2v[j]

    n_v = jnp.full((L,), n, jnp.float32)
    s1_v = jnp.full((L,), s1, jnp.float32)
    s2_v = jnp.full((L,), s2, jnp.float32)
    mean_v = s1_v / n_v
    var_v = (s2_v - s1_v * mean_v) / (n_v - 1.0)
    var_v = jnp.maximum(var_v, 1e-20)
    bits = lax.bitcast_convert_type(var_v, jnp.int32)
    r = lax.bitcast_convert_type(0x5F3759DF - (bits >> 1), jnp.float32)
    for _ in range(3):
        r = r * (1.5 - 0.5 * var_v * r * r)
    inv_v = 1.0 / (var_v * r + 1e-5)

    def chunk_body(ci, _):
        off = base + ci * CH
        pltpu.sync_copy(x_hbm.at[pl.ds(off, CH)], buf)

        def vec_body(i):
            v = buf[pl.ds(i, L)]
            buf[pl.ds(i, L)] = jnp.where(v >= 0.0, (v - mean_v) * inv_v, v)

        plsc.parallel_loop(0, CH, L, unroll=UNROLL)(vec_body)
        pltpu.sync_copy(buf, out_hbm.at[pl.ds(off, CH)])
        return 0

    lax.fori_loop(0, NCH, chunk_body, 0)


def kernel(x):
    xf = x.reshape(N)
    part = _stats_kernel(xf)
    out = _norm_kernel(xf, part)
    return out.reshape(x.shape)


# speedup vs baseline: 1.1433x; 1.1433x over previous
import functools

import jax
import jax.numpy as jnp
from jax import lax
from jax.experimental import pallas as pl
from jax.experimental.pallas import tpu as pltpu
from jax.experimental.pallas import tpu_sc as plsc

B = 8
E = 96 * 224 * 224
N = B * E
NC, NS, L = 2, 16, 16
NW = NC * NS
WPS = NW // B
EP = E // WPS
CH = 28672
NCH = EP // CH
NJ = NCH // 2
UNROLL = 8

_mesh = plsc.VectorSubcoreMesh(
    core_axis_name="c", subcore_axis_name="s", num_cores=NC, num_subcores=NS
)


def _worker_id():
    return lax.axis_index("s") * NC + lax.axis_index("c")


@functools.partial(
    pl.kernel,
    out_type=jax.ShapeDtypeStruct((NW, 3, L), jnp.float32),
    mesh=_mesh,
    scratch_types=[
        pltpu.VMEM((CH,), jnp.float32),
        pltpu.VMEM((CH,), jnp.float32),
        pltpu.VMEM((3, L), jnp.float32),
        pltpu.SemaphoreType.DMA,
        pltpu.SemaphoreType.DMA,
    ],
)
def _stats_kernel(x_hbm, part_hbm, buf_a, buf_b, part_v, sem_a, sem_b):
    wid = _worker_id()
    base = wid * EP

    def load(ci, buf, sem):
        return pltpu.make_async_copy(x_hbm.at[pl.ds(base + ci * CH, CH)], buf, sem)

    def chunk_stats(buf, acc):
        def vec_body(i, c):
            neg, s1, s2 = c
            v = buf[pl.ds(i, L)]
            m = jnp.maximum(v, 0.0)
            neg = neg + lax.shift_right_logical(
                lax.bitcast_convert_type(v, jnp.int32), 31
            )
            s1 = s1 + m
            s2 = s2 + m * m
            return (neg, s1, s2)

        return plsc.parallel_loop(0, CH, L, unroll=UNROLL, carry=acc)(vec_body)

    load(0, buf_a, sem_a).start()

    def pair_body(j, acc):
        load(2 * j + 1, buf_b, sem_b).start()
        load(2 * j, buf_a, sem_a).wait()
        acc = chunk_stats(buf_a, acc)

        @pl.when(j < NJ - 1)
        def _():
            load(2 * j + 2, buf_a, sem_a).start()

        load(2 * j + 1, buf_b, sem_b).wait()
        return chunk_stats(buf_b, acc)

    zf = jnp.zeros((L,), jnp.float32)
    zi = jnp.zeros((L,), jnp.int32)
    neg, s1, s2 = lax.fori_loop(0, NJ, pair_body, (zi, zf, zf))
    part_v[0, :] = neg.astype(jnp.float32)
    part_v[1, :] = s1
    part_v[2, :] = s2
    pltpu.sync_copy(part_v, part_hbm.at[wid])


@functools.partial(
    pl.kernel,
    out_type=jax.ShapeDtypeStruct((N,), jnp.float32),
    mesh=_mesh,
    scratch_types=[
        pltpu.VMEM((CH,), jnp.float32),
        pltpu.VMEM((CH,), jnp.float32),
        pltpu.VMEM((CH,), jnp.float32),
        pltpu.VMEM((CH,), jnp.float32),
        pltpu.VMEM((NW, 3, L), jnp.float32),
        pltpu.SemaphoreType.DMA,
        pltpu.SemaphoreType.DMA,
        pltpu.SemaphoreType.DMA,
        pltpu.SemaphoreType.DMA,
    ],
)
def _norm_kernel(
    x_hbm, part_hbm, out_hbm,
    in_a, in_b, out_a, out_b, part_v,
    lsem_a, lsem_b, ssem_a, ssem_b,
):
    wid = _worker_id()
    b = wid // WPS
    base = wid * EP

    pltpu.sync_copy(part_hbm, part_v)
    negv = jnp.zeros((L,), jnp.float32)
    s1v = jnp.zeros((L,), jnp.float32)
    s2v = jnp.zeros((L,), jnp.float32)
    for k in range(WPS):
        w = b * WPS + k
        negv = negv + part_v[w, 0, :]
        s1v = s1v + part_v[w, 1, :]
        s2v = s2v + part_v[w, 2, :]
    neg = negv[0]
    s1 = s1v[0]
    s2 = s2v[0]
    for j in range(1, L):
        neg = neg + negv[j]
        s1 = s1 + s1v[j]
        s2 = s2 + s2v[j]

    n_v = jnp.full((L,), float(E), jnp.float32) - jnp.full((L,), neg, jnp.float32)
    s1_v = jnp.full((L,), s1, jnp.float32)
    s2_v = jnp.full((L,), s2, jnp.float32)
    mean_v = s1_v / n_v
    var_v = (s2_v - s1_v * mean_v) / (n_v - 1.0)
    var_v = jnp.maximum(var_v, 1e-20)
    bits = lax.bitcast_convert_type(var_v, jnp.int32)
    r = lax.bitcast_convert_type(0x5F3759DF - (bits >> 1), jnp.float32)
    for _ in range(3):
        r = r * (1.5 - 0.5 * var_v * r * r)
    inv_v = 1.0 / (var_v * r + 1e-5)
    c_v = -mean_v * inv_v

    def load(ci, buf, sem):
        return pltpu.make_async_copy(x_hbm.at[pl.ds(base + ci * CH, CH)], buf, sem)

    def store(ci, buf, sem):
        return pltpu.make_async_copy(buf, out_hbm.at[pl.ds(base + ci * CH, CH)], sem)

    def chunk_norm(ibuf, obuf):
        def vec_body(i):
            v = ibuf[pl.ds(i, L)]
            obuf[pl.ds(i, L)] = jnp.where(v >= 0.0, v * inv_v + c_v, v)

        plsc.parallel_loop(0, CH, L, unroll=UNROLL)(vec_body)

    load(0, in_a, lsem_a).start()

    def pair_body(j, carry):
        load(2 * j + 1, in_b, lsem_b).start()
        load(2 * j, in_a, lsem_a).wait()

        @pl.when(j > 0)
        def _():
            store(2 * j - 2, out_a, ssem_a).wait()

        chunk_norm(in_a, out_a)
        store(2 * j, out_a, ssem_a).start()

        @pl.when(j < NJ - 1)
        def _():
            load(2 * j + 2, in_a, lsem_a).start()

        load(2 * j + 1, in_b, lsem_b).wait()

        @pl.when(j > 0)
        def _():
            store(2 * j - 1, out_b, ssem_b).wait()

        chunk_norm(in_b, out_b)
        store(2 * j + 1, out_b, ssem_b).start()
        return carry

    lax.fori_loop(0, NJ, pair_body, 0)
    store(NCH - 2, out_a, ssem_a).wait()
    store(NCH - 1, out_b, ssem_b).wait()


def kernel(x):
    xf = x.reshape(N)
    part = _stats_kernel(xf)
    out = _norm_kernel(xf, part)
    return out.reshape(x.shape)


# speedup vs baseline: 1.3614x; 1.1907x over previous
import functools

import jax
import jax.numpy as jnp
from jax import lax
from jax.experimental import pallas as pl
from jax.experimental.pallas import tpu as pltpu
from jax.experimental.pallas import tpu_sc as plsc

B = 8
C = 96
W = 224
RG = C * (W // 8)
T = B * RG
E = C * W * W
NC, NS, L = 2, 16, 16
NW = NC * NS
WPS = NW // B
TPW = T // NW
KV = W // L

TB1 = 24
NJ1 = TPW // TB1 // 2
TB2 = 12
NJ2 = TPW // TB2 // 2

_mesh = plsc.VectorSubcoreMesh(
    core_axis_name="c", subcore_axis_name="s", num_cores=NC, num_subcores=NS
)


def _worker_id():
    return lax.axis_index("s") * NC + lax.axis_index("c")


@functools.partial(
    pl.kernel,
    out_type=jax.ShapeDtypeStruct((NW, 3, L), jnp.float32),
    mesh=_mesh,
    scratch_types=[
        pltpu.VMEM((TB1, 8, W), jnp.float32),
        pltpu.VMEM((TB1, 8, W), jnp.float32),
        pltpu.VMEM((3, L), jnp.float32),
        pltpu.SemaphoreType.DMA,
        pltpu.SemaphoreType.DMA,
    ],
)
def _stats_kernel(x_hbm, part_hbm, buf_a, buf_b, part_v, sem_a, sem_b):
    wid = _worker_id()
    base = wid * TPW

    def load(ci, buf, sem):
        return pltpu.make_async_copy(
            x_hbm.at[pl.ds(base + ci * TB1, TB1)], buf, sem
        )

    def chunk_stats(buf, acc):
        def rg_body(t, c):
            neg, s1, s2 = c
            for r in range(8):
                for k in range(KV):
                    v = buf[t, r, pl.ds(k * L, L)]
                    m = jnp.maximum(v, 0.0)
                    neg = neg + lax.shift_right_logical(
                        lax.bitcast_convert_type(v, jnp.int32), 31
                    )
                    s1 = s1 + m
                    s2 = s2 + m * m
            return (neg, s1, s2)

        return plsc.parallel_loop(0, TB1, 1, carry=acc)(rg_body)

    load(0, buf_a, sem_a).start()

    def pair_body(j, acc):
        load(2 * j + 1, buf_b, sem_b).start()
        load(2 * j, buf_a, sem_a).wait()
        acc = chunk_stats(buf_a, acc)

        @pl.when(j < NJ1 - 1)
        def _():
            load(2 * j + 2, buf_a, sem_a).start()

        load(2 * j + 1, buf_b, sem_b).wait()
        return chunk_stats(buf_b, acc)

    zf = jnp.zeros((L,), jnp.float32)
    zi = jnp.zeros((L,), jnp.int32)
    neg, s1, s2 = lax.fori_loop(0, NJ1, pair_body, (zi, zf, zf))
    part_v[0, :] = neg.astype(jnp.float32)
    part_v[1, :] = s1
    part_v[2, :] = s2
    pltpu.sync_copy(part_v, part_hbm.at[wid])


@functools.partial(
    pl.kernel,
    out_type=jax.ShapeDtypeStruct((T, 8, W), jnp.float32),
    mesh=_mesh,
    scratch_types=[
        pltpu.VMEM((TB2, 8, W), jnp.float32),
        pltpu.VMEM((TB2, 8, W), jnp.float32),
        pltpu.VMEM((TB2, 8, W), jnp.float32),
        pltpu.VMEM((TB2, 8, W), jnp.float32),
        pltpu.VMEM((NW, 3, L), jnp.float32),
        pltpu.SemaphoreType.DMA,
        pltpu.SemaphoreType.DMA,
        pltpu.SemaphoreType.DMA,
        pltpu.SemaphoreType.DMA,
    ],
)
def _norm_kernel(
    x_hbm, part_hbm, out_hbm,
    in_a, in_b, out_a, out_b, part_v,
    lsem_a, lsem_b, ssem_a, ssem_b,
):
    wid = _worker_id()
    b = wid // WPS
    base = wid * TPW

    pltpu.sync_copy(part_hbm, part_v)
    negv = jnp.zeros((L,), jnp.float32)
    s1v = jnp.zeros((L,), jnp.float32)
    s2v = jnp.zeros((L,), jnp.float32)
    for k in range(WPS):
        w = b * WPS + k
        negv = negv + part_v[w, 0, :]
        s1v = s1v + part_v[w, 1, :]
        s2v = s2v + part_v[w, 2, :]
    neg = negv[0]
    s1 = s1v[0]
    s2 = s2v[0]
    for j in range(1, L):
        neg = neg + negv[j]
        s1 = s1 + s1v[j]
        s2 = s2 + s2v[j]

    n_v = jnp.full((L,), float(E), jnp.float32) - jnp.full((L,), neg, jnp.float32)
    s1_v = jnp.full((L,), s1, jnp.float32)
    s2_v = jnp.full((L,), s2, jnp.float32)
    mean_v = s1_v / n_v
    var_v = (s2_v - s1_v * mean_v) / (n_v - 1.0)
    var_v = jnp.maximum(var_v, 1e-20)
    bits = lax.bitcast_convert_type(var_v, jnp.int32)
    r = lax.bitcast_convert_type(0x5F3759DF - (bits >> 1), jnp.float32)
    for _ in range(3):
        r = r * (1.5 - 0.5 * var_v * r * r)
    inv_v = 1.0 / (var_v * r + 1e-5)
    c_v = -mean_v * inv_v

    def load(ci, buf, sem):
        return pltpu.make_async_copy(
            x_hbm.at[pl.ds(base + ci * TB2, TB2)], buf, sem
        )

    def store(ci, buf, sem):
        return pltpu.make_async_copy(
            buf, out_hbm.at[pl.ds(base + ci * TB2, TB2)], sem
        )

    def chunk_norm(ibuf, obuf):
        def rg_body(t):
            for r in range(8):
                for k in range(KV):
                    v = ibuf[t, r, pl.ds(k * L, L)]
                    obuf[t, r, pl.ds(k * L, L)] = jnp.where(
                        v >= 0.0, v * inv_v + c_v, v
                    )

        plsc.parallel_loop(0, TB2, 1)(rg_body)

    load(0, in_a, lsem_a).start()

    def pair_body(j, carry):
        load(2 * j + 1, in_b, lsem_b).start()
        load(2 * j, in_a, lsem_a).wait()

        @pl.when(j > 0)
        def _():
            store(2 * j - 2, out_a, ssem_a).wait()

        chunk_norm(in_a, out_a)
        store(2 * j, out_a, ssem_a).start()

        @pl.when(j < NJ2 - 1)
        def _():
            load(2 * j + 2, in_a, lsem_a).start()

        load(2 * j + 1, in_b, lsem_b).wait()

        @pl.when(j > 0)
        def _():
            store(2 * j - 1, out_b, ssem_b).wait()

        chunk_norm(in_b, out_b)
        store(2 * j + 1, out_b, ssem_b).start()
        return carry

    lax.fori_loop(0, NJ2, pair_body, 0)
    store(2 * NJ2 - 2, out_a, ssem_a).wait()
    store(2 * NJ2 - 1, out_b, ssem_b).wait()


def kernel(x):
    x3 = x.reshape(T, 8, W)
    part = _stats_kernel(x3)
    out = _norm_kernel(x3, part)
    return out.reshape(x.shape)


# speedup vs baseline: 4.1355x; 3.0378x over previous
import functools

import jax
import jax.numpy as jnp
from jax import lax
from jax.experimental import pallas as pl
from jax.experimental.pallas import tpu as pltpu
from jax.experimental.pallas import tpu_sc as plsc

B = 8
C = 96
W = 224
R = B * C * W
E = C * W * W
NC, NS, L = 2, 16, 16
NW = NC * NS
WPS = NW // B
RPW = R // NW
KV = W // L

RB1 = 192
NJ1 = RPW // RB1 // 2
RB2 = 96
NJ2 = RPW // RB2 // 2

_mesh = plsc.VectorSubcoreMesh(
    core_axis_name="c", subcore_axis_name="s", num_cores=NC, num_subcores=NS
)


def _worker_id():
    return lax.axis_index("s") * NC + lax.axis_index("c")


@functools.partial(
    pl.kernel,
    out_type=jax.ShapeDtypeStruct((NW, 3, L), jnp.float32),
    mesh=_mesh,
    scratch_types=[
        pltpu.VMEM((RB1, W), jnp.float32),
        pltpu.VMEM((RB1, W), jnp.float32),
        pltpu.VMEM((3, L), jnp.float32),
        pltpu.SemaphoreType.DMA,
        pltpu.SemaphoreType.DMA,
    ],
)
def _stats_kernel(x_hbm, part_hbm, buf_a, buf_b, part_v, sem_a, sem_b):
    wid = _worker_id()
    base = wid * RPW

    def load(ci, buf, sem):
        return pltpu.make_async_copy(
            x_hbm.at[pl.ds(base + ci * RB1, RB1)], buf, sem
        )

    def chunk_stats(buf, acc):
        def row_body(j, c):
            neg, s1, s2 = c
            for k in range(KV):
                v = buf[j, pl.ds(k * L, L)]
                m = jnp.maximum(v, 0.0)
                neg = neg + lax.shift_right_logical(
                    lax.bitcast_convert_type(v, jnp.int32), 31
                )
                s1 = s1 + m
                s2 = s2 + m * m
            return (neg, s1, s2)

        return plsc.parallel_loop(0, RB1, 1, carry=acc)(row_body)

    load(0, buf_a, sem_a).start()

    def pair_body(j, acc):
        load(2 * j + 1, buf_b, sem_b).start()
        load(2 * j, buf_a, sem_a).wait()
        acc = chunk_stats(buf_a, acc)

        @pl.when(j < NJ1 - 1)
        def _():
            load(2 * j + 2, buf_a, sem_a).start()

        load(2 * j + 1, buf_b, sem_b).wait()
        return chunk_stats(buf_b, acc)

    zf = jnp.zeros((L,), jnp.float32)
    zi = jnp.zeros((L,), jnp.int32)
    neg, s1, s2 = lax.fori_loop(0, NJ1, pair_body, (zi, zf, zf))
    part_v[0, :] = neg.astype(jnp.float32)
    part_v[1, :] = s1
    part_v[2, :] = s2
    pltpu.sync_copy(part_v, part_hbm.at[wid])


@functools.partial(
    pl.kernel,
    out_type=jax.ShapeDtypeStruct((R, W), jnp.float32),
    mesh=_mesh,
    scratch_types=[
        pltpu.VMEM((RB2, W), jnp.float32),
        pltpu.VMEM((RB2, W), jnp.float32),
        pltpu.VMEM((RB2, W), jnp.float32),
        pltpu.VMEM((RB2, W), jnp.float32),
        pltpu.VMEM((NW, 3, L), jnp.float32),
        pltpu.SemaphoreType.DMA,
        pltpu.SemaphoreType.DMA,
        pltpu.SemaphoreType.DMA,
        pltpu.SemaphoreType.DMA,
    ],
)
def _norm_kernel(
    x_hbm, part_hbm, out_hbm,
    in_a, in_b, out_a, out_b, part_v,
    lsem_a, lsem_b, ssem_a, ssem_b,
):
    wid = _worker_id()
    b = wid // WPS
    base = wid * RPW

    pltpu.sync_copy(part_hbm, part_v)
    negv = jnp.zeros((L,), jnp.float32)
    s1v = jnp.zeros((L,), jnp.float32)
    s2v = jnp.zeros((L,), jnp.float32)
    for k in range(WPS):
        w = b * WPS + k
        negv = negv + part_v[w, 0, :]
        s1v = s1v + part_v[w, 1, :]
        s2v = s2v + part_v[w, 2, :]
    neg = negv[0]
    s1 = s1v[0]
    s2 = s2v[0]
    for j in range(1, L):
        neg = neg + negv[j]
        s1 = s1 + s1v[j]
        s2 = s2 + s2v[j]

    n_v = jnp.full((L,), float(E), jnp.float32) - jnp.full((L,), neg, jnp.float32)
    s1_v = jnp.full((L,), s1, jnp.float32)
    s2_v = jnp.full((L,), s2, jnp.float32)
    mean_v = s1_v / n_v
    var_v = (s2_v - s1_v * mean_v) / (n_v - 1.0)
    var_v = jnp.maximum(var_v, 1e-20)
    bits = lax.bitcast_convert_type(var_v, jnp.int32)
    r = lax.bitcast_convert_type(0x5F3759DF - (bits >> 1), jnp.float32)
    for _ in range(3):
        r = r * (1.5 - 0.5 * var_v * r * r)
    inv_v = 1.0 / (var_v * r + 1e-5)
    c_v = -mean_v * inv_v

    def load(ci, buf, sem):
        return pltpu.make_async_copy(
            x_hbm.at[pl.ds(base + ci * RB2, RB2)], buf, sem
        )

    def store(ci, buf, sem):
        return pltpu.make_async_copy(
            buf, out_hbm.at[pl.ds(base + ci * RB2, RB2)], sem
        )

    def chunk_norm(ibuf, obuf):
        def row_body(j):
            for k in range(KV):
                v = ibuf[j, pl.ds(k * L, L)]
                obuf[j, pl.ds(k * L, L)] = jnp.where(
                    v >= 0.0, v * inv_v + c_v, v
                )

        plsc.parallel_loop(0, RB2, 1)(row_body)

    load(0, in_a, lsem_a).start()

    def pair_body(j, carry):
        load(2 * j + 1, in_b, lsem_b).start()
        load(2 * j, in_a, lsem_a).wait()

        @pl.when(j > 0)
        def _():
            store(2 * j - 2, out_a, ssem_a).wait()

        chunk_norm(in_a, out_a)
        store(2 * j, out_a, ssem_a).start()

        @pl.when(j < NJ2 - 1)
        def _():
            load(2 * j + 2, in_a, lsem_a).start()

        load(2 * j + 1, in_b, lsem_b).wait()

        @pl.when(j > 0)
        def _():
            store(2 * j - 1, out_b, ssem_b).wait()

        chunk_norm(in_b, out_b)
        store(2 * j + 1, out_b, ssem_b).start()
        return carry

    lax.fori_loop(0, NJ2, pair_body, 0)
    store(2 * NJ2 - 2, out_a, ssem_a).wait()
    store(2 * NJ2 - 1, out_b, ssem_b).wait()


def kernel(x):
    x2 = x.reshape(R, W)
    part = _stats_kernel(x2)
    out = _norm_kernel(x2, part)
    return out.reshape(x.shape)
